# baseline (speedup 1.0000x reference)
import functools

import jax
import jax.numpy as jnp
import numpy as np
from jax import lax
from jax.experimental import pallas as pl
from jax.experimental.pallas import tpu as pltpu
from jax.experimental.pallas import tpu_sc as plsc

N = 10000
H = 16
NC = 2
NS = 16
NW = NC * NS
C = 128
NPAD = 10240
RPS = NPAD // NS
DUMP = N
NP8 = N // 8
NPAD8 = NPAD // 8

_mesh = plsc.VectorSubcoreMesh(core_axis_name="c", subcore_axis_name="s")
_sc_params = pltpu.CompilerParams(use_tc_tiling_on_sc=False)


def _np_f32(a):
    return jnp.asarray(np.asarray(a, dtype=np.float32))


_EYE8 = np.eye(8, dtype=np.float32)
_SX = np.zeros((128, 16), dtype=np.float32)
for _g in range(8):
    for _c in range(2):
        _SX[16 * _g + _c, 2 * _g + _c] = 1.0
_PSWAP = np.kron(_EYE8, np.array([[0, 1], [1, 0]], dtype=np.float32))
_PPAIR = np.kron(_EYE8, np.ones((2, 2), dtype=np.float32))


def _sc_hist(dst3, nchunks):
    nsem = 8
    assert nchunks % nsem == 0

    @functools.partial(
        pl.kernel,
        out_type=jax.ShapeDtypeStruct((NC, NPAD, H), jnp.float32),
        mesh=_mesh,
        compiler_params=_sc_params,
        scratch_types=[
            pltpu.VMEM((nchunks, C), jnp.int32),
            pltpu.VMEM((C, H), jnp.float32),
            pltpu.VMEM_SHARED((NPAD, H), jnp.float32),
        ] + [pltpu.SemaphoreType.DMA] * (nsem + 1),
    )
    def k(dst_hbm, zer_hbm, out_hbm, dst_v, ones_v, acc, *sems):
        psem = sems[nsem]
        c = lax.axis_index("c")
        s = lax.axis_index("s")
        wid = c * NS + s

        @pl.loop(0, C)
        def _(i):
            ones_v[i, :] = jnp.ones((H,), jnp.float32)

        z = pltpu.async_copy(
            zer_hbm.at[pl.ds(s * RPS, RPS)], acc.at[pl.ds(s * RPS, RPS)], psem)
        i1 = pltpu.async_copy(dst_hbm.at[wid], dst_v, psem)
        z.wait()
        i1.wait()
        plsc.subcore_barrier()

        def scatter(j, b):
            pltpu.async_copy(ones_v, acc.at[dst_v.at[j]], sems[b], add=True)

        def wait_scatter(j, b):
            pltpu.make_async_copy(ones_v, acc.at[dst_v.at[j]], sems[b]).wait()

        for b in range(nsem):
            scatter(b, b)

        @pl.loop(1, nchunks // nsem)
        def _(t):
            j0 = (t - 1) * nsem
            for b in range(nsem):
                wait_scatter(j0 + b, b)
                scatter(t * nsem + b, b)

        for b in range(nsem):
            wait_scatter(nchunks - nsem + b, b)

        plsc.subcore_barrier()
        pltpu.sync_copy(acc.at[pl.ds(s * RPS, RPS)],
                        out_hbm.at[c, pl.ds(s * RPS, RPS)])

    return k(dst3, jnp.zeros((NPAD, H), jnp.float32))


def _sc_agg(table, src3, dst3, nchunks):
    nbuf = 8
    assert nchunks % nbuf == 0

    @functools.partial(
        pl.kernel,
        out_type=jax.ShapeDtypeStruct((NC, NPAD, H), jnp.float32),
        mesh=_mesh,
        compiler_params=_sc_params,
        scratch_types=[
            pltpu.VMEM((nchunks, C), jnp.int32),
            pltpu.VMEM((nchunks, C), jnp.int32),
            pltpu.VMEM((nbuf, C, H), jnp.float32),
            pltpu.VMEM_SHARED((NPAD, H), jnp.float32),
            pltpu.VMEM_SHARED((NPAD, H), jnp.float32),
        ] + [pltpu.SemaphoreType.DMA] * (2 * nbuf + 1),
    )
    def k(table_hbm, zer_hbm, src_hbm, dst_hbm, out_hbm, src_v, dst_v, rows_v,
          acc, tab_s, *sems):
        gsem = sems[:nbuf]
        ssem = sems[nbuf:2 * nbuf]
        psem = sems[2 * nbuf]
        c = lax.axis_index("c")
        s = lax.axis_index("s")
        wid = c * NS + s

        z = pltpu.async_copy(
            zer_hbm.at[pl.ds(s * RPS, RPS)], acc.at[pl.ds(s * RPS, RPS)], psem)
        t0 = pltpu.async_copy(
            table_hbm.at[pl.ds(s * RPS, RPS)], tab_s.at[pl.ds(s * RPS, RPS)],
            psem)
        i0 = pltpu.async_copy(src_hbm.at[wid], src_v, psem)
        i1 = pltpu.async_copy(dst_hbm.at[wid], dst_v, psem)
        z.wait()
        t0.wait()
        i0.wait()
        i1.wait()
        plsc.subcore_barrier()

        def gather(j, b):
            pltpu.async_copy(tab_s.at[src_v.at[j]], rows_v.at[b], gsem[b])

        def wait_gather(j, b):
            pltpu.make_async_copy(
                tab_s.at[src_v.at[j]], rows_v.at[b], gsem[b]).wait()

        def scatter(j, b):
            pltpu.async_copy(
                rows_v.at[b], acc.at[dst_v.at[j]], ssem[b], add=True)

        def wait_scatter(j, b):
            pltpu.make_async_copy(
                rows_v.at[b], acc.at[dst_v.at[j]], ssem[b]).wait()

        for b in range(nbuf):
            gather(b, b)

        @pl.loop(1, nchunks // nbuf)
        def _(t):
            j0 = (t - 1) * nbuf
            for b in range(nbuf):
                wait_gather(j0 + b, b)
                scatter(j0 + b, b)
            for b in range(nbuf):
                wait_scatter(j0 + b, b)
                gather(t * nbuf + b, b)

        j0 = nchunks - nbuf
        for b in range(nbuf):
            wait_gather(j0 + b, b)
            scatter(j0 + b, b)
        for b in range(nbuf):
            wait_scatter(j0 + b, b)

        plsc.subcore_barrier()
        pltpu.sync_copy(acc.at[pl.ds(s * RPS, RPS)],
                        out_hbm.at[c, pl.ds(s * RPS, RPS)])

    return k(table, jnp.zeros((NPAD, H), jnp.float32), src3, dst3)


def _tc_matmul1(x8, w1k):
    def body(x_ref, w_ref, o_ref):
        d = jnp.dot(x_ref[...], w_ref[...], preferred_element_type=jnp.float32)
        o_ref[...] = jnp.concatenate(
            [d, jnp.zeros((NPAD8 - NP8, 128), jnp.float32)], axis=0)

    return pl.pallas_call(
        body,
        out_shape=jax.ShapeDtypeStruct((NPAD8, 128), jnp.float32),
    )(x8, w1k)


def _tc_prep(h1p, histp):
    def body(h_ref, p_ref, hs_ref, dinv_ref):
        deg = p_ref[0] + p_ref[1] + 1.0
        dinv = lax.rsqrt(deg)
        dinv_ref[...] = dinv
        hs_ref[...] = h_ref[...] * dinv

    return pl.pallas_call(
        body,
        out_shape=(jax.ShapeDtypeStruct((NPAD8, 128), jnp.float32),
                   jax.ShapeDtypeStruct((NPAD8, 128), jnp.float32)),
    )(h1p, histp)


def _tc_mid(pp, hsp, dinvp, bt, wk):
    def body(p_ref, hs_ref, dinv_ref, b_ref, w_ref, o_ref):
        psum = p_ref[0] + p_ref[1]
        z = jax.nn.relu(dinv_ref[...] * (psum + hs_ref[...]) + b_ref[...])
        o_ref[...] = jnp.dot(z, w_ref[...],
                             preferred_element_type=jnp.float32) * dinv_ref[...]

    return pl.pallas_call(
        body,
        out_shape=jax.ShapeDtypeStruct((NPAD8, 128), jnp.float32),
    )(pp, hsp, dinvp, bt.reshape(1, 128), wk)


def _tc_final(pp, hs3p, dinvp, b2t, sx, pswap, ppair):
    def body(p_ref, hs_ref, dinv_ref, b_ref, sx_ref, sw_ref, pr_ref, o_ref):
        zp = dinv_ref[:NP8] * (p_ref[0, :NP8] + p_ref[1, :NP8]
                               + hs_ref[:NP8])
        z2 = jnp.dot(zp, sx_ref[...],
                     preferred_element_type=jnp.float32) + b_ref[...]
        zsw = jnp.dot(z2, sw_ref[...], preferred_element_type=jnp.float32)
        m = jnp.maximum(z2, zsw)
        sh = z2 - m
        es = jnp.dot(jnp.exp(sh), pr_ref[...],
                     preferred_element_type=jnp.float32)
        o_ref[...] = sh - jnp.log(es)

    return pl.pallas_call(
        body,
        out_shape=jax.ShapeDtypeStruct((NP8, 16), jnp.float32),
    )(pp, hs3p, dinvp, b2t.reshape(1, 16), sx, pswap, ppair)


def kernel(x, edge_index, W1, b1, Wm, bm, W2, b2):
    e = edge_index.shape[1]
    nchunks = -(-e // (NW * C))
    nchunks = -(-nchunks // 16) * 16
    epad = NW * C * nchunks
    dst3 = jnp.pad(edge_index[1].astype(jnp.int32), (0, epad - e),
                   constant_values=DUMP).reshape(NW, nchunks, C)
    src3 = jnp.pad(edge_index[0].astype(jnp.int32), (0, epad - e),
                   constant_values=DUMP).reshape(NW, nchunks, C)

    x8 = x.reshape(NP8, 8 * x.shape[1])
    w1k = jnp.kron(_np_f32(_EYE8), W1)
    wmk = jnp.kron(_np_f32(_EYE8), Wm)
    w2pad = jnp.zeros((H, H), jnp.float32).at[:, :2].set(W2)
    w2k = jnp.kron(_np_f32(_EYE8), w2pad)
    b1t = jnp.tile(b1, 8)
    bmt = jnp.tile(bm, 8)
    b2t = jnp.tile(b2, 8)
    sx = _np_f32(_SX)
    pswap = _np_f32(_PSWAP)
    ppair = _np_f32(_PPAIR)

    hist = _sc_hist(dst3, nchunks)
    histp = hist.reshape(NC, NPAD8, 128)
    h1p = _tc_matmul1(x8, w1k)
    hs1p, dinvp = _tc_prep(h1p, histp)
    p1 = _sc_agg(hs1p.reshape(NPAD, H), src3, dst3, nchunks)
    hs2p = _tc_mid(p1.reshape(NC, NPAD8, 128), hs1p, dinvp, b1t, wmk)
    p2 = _sc_agg(hs2p.reshape(NPAD, H), src3, dst3, nchunks)
    hs3p = _tc_mid(p2.reshape(NC, NPAD8, 128), hs2p, dinvp, bmt, w2k)
    p3 = _sc_agg(hs3p.reshape(NPAD, H), src3, dst3, nchunks)
    out2 = _tc_final(p3.reshape(NC, NPAD8, 128), hs3p, dinvp, b2t,
                     sx, pswap, ppair)
    return out2.reshape(N, 2)


# speedup vs baseline: 1.0806x; 1.0806x over previous
import functools

import jax
import jax.numpy as jnp
import numpy as np
from jax import lax
from jax.experimental import pallas as pl
from jax.experimental.pallas import tpu as pltpu
from jax.experimental.pallas import tpu_sc as plsc

N = 10000
H = 16
NC = 2
NS = 16
NW = NC * NS
C = 128
NPAD = 10240
RPS = NPAD // NS
DUMP = N
NP8 = N // 8
NPAD8 = NPAD // 8

_mesh = plsc.VectorSubcoreMesh(core_axis_name="c", subcore_axis_name="s")
_sc_params = pltpu.CompilerParams(use_tc_tiling_on_sc=False)


def _np_f32(a):
    return jnp.asarray(np.asarray(a, dtype=np.float32))


_EYE8 = np.eye(8, dtype=np.float32)
_SX = np.zeros((128, 16), dtype=np.float32)
for _g in range(8):
    for _c in range(2):
        _SX[16 * _g + _c, 2 * _g + _c] = 1.0
_PSWAP = np.kron(_EYE8, np.array([[0, 1], [1, 0]], dtype=np.float32))
_PPAIR = np.kron(_EYE8, np.ones((2, 2), dtype=np.float32))


def _sc_hist(dst3, nchunks):
    nsem = 8
    assert nchunks % nsem == 0

    @functools.partial(
        pl.kernel,
        out_type=jax.ShapeDtypeStruct((NC, NPAD, H), jnp.float32),
        mesh=_mesh,
        compiler_params=_sc_params,
        scratch_types=[
            pltpu.VMEM((nchunks, C), jnp.int32),
            pltpu.VMEM((C, H), jnp.float32),
            pltpu.VMEM_SHARED((NPAD, H), jnp.float32),
        ] + [pltpu.SemaphoreType.DMA] * (nsem + 1),
    )
    def k(dst_hbm, zer_hbm, out_hbm, dst_v, ones_v, acc, *sems):
        psem = sems[nsem]
        c = lax.axis_index("c")
        s = lax.axis_index("s")
        wid = c * NS + s

        @pl.loop(0, C)
        def _(i):
            ones_v[i, :] = jnp.ones((H,), jnp.float32)

        z = pltpu.async_copy(
            zer_hbm.at[pl.ds(s * RPS, RPS)], acc.at[pl.ds(s * RPS, RPS)], psem)
        i1 = pltpu.async_copy(dst_hbm.at[wid], dst_v, psem)
        z.wait()
        i1.wait()
        plsc.subcore_barrier()

        def scatter(j, b):
            pltpu.async_copy(ones_v, acc.at[dst_v.at[j]], sems[b], add=True)

        def wait_scatter(j, b):
            pltpu.make_async_copy(ones_v, acc.at[dst_v.at[j]], sems[b]).wait()

        for b in range(nsem):
            scatter(b, b)

        @pl.loop(1, nchunks // nsem)
        def _(t):
            j0 = (t - 1) * nsem
            for b in range(nsem):
                wait_scatter(j0 + b, b)
                scatter(t * nsem + b, b)

        for b in range(nsem):
            wait_scatter(nchunks - nsem + b, b)

        plsc.subcore_barrier()
        pltpu.sync_copy(acc.at[pl.ds(s * RPS, RPS)],
                        out_hbm.at[c, pl.ds(s * RPS, RPS)])

    return k(dst3, jnp.zeros((NPAD, H), jnp.float32))


def _sc_agg(table, src3, dst3, nchunks):
    nbuf = 8
    assert nchunks % nbuf == 0

    @functools.partial(
        pl.kernel,
        out_type=jax.ShapeDtypeStruct((NC, NPAD, H), jnp.float32),
        mesh=_mesh,
        compiler_params=_sc_params,
        scratch_types=[
            pltpu.VMEM((nchunks, C), jnp.int32),
            pltpu.VMEM((nchunks, C), jnp.int32),
            pltpu.VMEM((nbuf, C, H), jnp.float32),
            pltpu.VMEM_SHARED((NPAD, H), jnp.float32),
            pltpu.VMEM_SHARED((NPAD, H), jnp.float32),
        ] + [pltpu.SemaphoreType.DMA] * (2 * nbuf + 1),
    )
    def k(table_hbm, zer_hbm, src_hbm, dst_hbm, out_hbm, src_v, dst_v, rows_v,
          acc, tab_s, *sems):
        gsem = sems[:nbuf]
        ssem = sems[nbuf:2 * nbuf]
        psem = sems[2 * nbuf]
        c = lax.axis_index("c")
        s = lax.axis_index("s")
        wid = c * NS + s

        z = pltpu.async_copy(
            zer_hbm.at[pl.ds(s * RPS, RPS)], acc.at[pl.ds(s * RPS, RPS)], psem)
        t0 = pltpu.async_copy(
            table_hbm.at[pl.ds(s * RPS, RPS)], tab_s.at[pl.ds(s * RPS, RPS)],
            psem)
        i0 = pltpu.async_copy(src_hbm.at[wid], src_v, psem)
        i1 = pltpu.async_copy(dst_hbm.at[wid], dst_v, psem)
        z.wait()
        t0.wait()
        i0.wait()
        i1.wait()
        plsc.subcore_barrier()

        def gather(j, b):
            pltpu.async_copy(tab_s.at[src_v.at[j]], rows_v.at[b], gsem[b])

        def wait_gather(j, b):
            pltpu.make_async_copy(
                tab_s.at[src_v.at[j]], rows_v.at[b], gsem[b]).wait()

        def scatter(j, b):
            pltpu.async_copy(
                rows_v.at[b], acc.at[dst_v.at[j]], ssem[b], add=True)

        def wait_scatter(j, b):
            pltpu.make_async_copy(
                rows_v.at[b], acc.at[dst_v.at[j]], ssem[b]).wait()

        for b in range(nbuf):
            gather(b, b)

        @pl.loop(1, nchunks // nbuf)
        def _(t):
            j0 = (t - 1) * nbuf
            for b in range(nbuf):
                wait_gather(j0 + b, b)
                scatter(j0 + b, b)
            for b in range(nbuf):
                wait_scatter(j0 + b, b)
                gather(t * nbuf + b, b)

        j0 = nchunks - nbuf
        for b in range(nbuf):
            wait_gather(j0 + b, b)
            scatter(j0 + b, b)
        for b in range(nbuf):
            wait_scatter(j0 + b, b)

        plsc.subcore_barrier()
        pltpu.sync_copy(acc.at[pl.ds(s * RPS, RPS)],
                        out_hbm.at[c, pl.ds(s * RPS, RPS)])

    return k(table, jnp.zeros((NPAD, H), jnp.float32), src3, dst3)


def _tc_matmul1(x8, w1k):
    def body(x_ref, w_ref, o_ref):
        d = jnp.dot(x_ref[...], w_ref[...], preferred_element_type=jnp.float32)
        o_ref[...] = jnp.concatenate(
            [d, jnp.zeros((NPAD8 - NP8, 128), jnp.float32)], axis=0)

    return pl.pallas_call(
        body,
        out_shape=jax.ShapeDtypeStruct((NPAD8, 128), jnp.float32),
    )(x8, w1k)


def _tc_prep(h1p, histp):
    def body(h_ref, p_ref, hs_ref, dinv_ref):
        deg = p_ref[0] + p_ref[1] + 1.0
        dinv = lax.rsqrt(deg)
        dinv_ref[...] = dinv
        hs_ref[...] = h_ref[...] * dinv

    return pl.pallas_call(
        body,
        out_shape=(jax.ShapeDtypeStruct((NPAD8, 128), jnp.float32),
                   jax.ShapeDtypeStruct((NPAD8, 128), jnp.float32)),
    )(h1p, histp)


def _tc_mid(pp, hsp, dinvp, bt, wk):
    def body(p_ref, hs_ref, dinv_ref, b_ref, w_ref, o_ref):
        psum = p_ref[0] + p_ref[1]
        z = jax.nn.relu(dinv_ref[...] * (psum + hs_ref[...]) + b_ref[...])
        o_ref[...] = jnp.dot(z, w_ref[...],
                             preferred_element_type=jnp.float32) * dinv_ref[...]

    return pl.pallas_call(
        body,
        out_shape=jax.ShapeDtypeStruct((NPAD8, 128), jnp.float32),
    )(pp, hsp, dinvp, bt.reshape(1, 128), wk)


def _tc_final(pp, hs3p, dinvp, b2t, sx, pswap, ppair):
    def body(p_ref, hs_ref, dinv_ref, b_ref, sx_ref, sw_ref, pr_ref, o_ref):
        zp = dinv_ref[:NP8] * (p_ref[0, :NP8] + p_ref[1, :NP8]
                               + hs_ref[:NP8])
        z2 = jnp.dot(zp, sx_ref[...],
                     preferred_element_type=jnp.float32) + b_ref[...]
        zsw = jnp.dot(z2, sw_ref[...], preferred_element_type=jnp.float32)
        m = jnp.maximum(z2, zsw)
        sh = z2 - m
        es = jnp.dot(jnp.exp(sh), pr_ref[...],
                     preferred_element_type=jnp.float32)
        o_ref[...] = sh - jnp.log(es)

    return pl.pallas_call(
        body,
        out_shape=jax.ShapeDtypeStruct((NP8, 16), jnp.float32),
    )(pp, hs3p, dinvp, b2t.reshape(1, 16), sx, pswap, ppair)


def kernel(x, edge_index, W1, b1, Wm, bm, W2, b2):
    e = edge_index.shape[1]
    nchunks = -(-e // (NW * C))
    nchunks = -(-nchunks // 16) * 16
    epad = NW * C * nchunks
    ei = jnp.pad(edge_index.astype(jnp.int32), ((0, 0), (0, epad - e)),
                 constant_values=DUMP).reshape(2, NW, nchunks, C)
    src3 = ei[0]
    dst3 = ei[1]

    x8 = x.reshape(NP8, 8 * x.shape[1])
    w1k = jnp.kron(_np_f32(_EYE8), W1)
    wmk = jnp.kron(_np_f32(_EYE8), Wm)
    w2pad = jnp.zeros((H, H), jnp.float32).at[:, :2].set(W2)
    w2k = jnp.kron(_np_f32(_EYE8), w2pad)
    b1t = jnp.tile(b1, 8)
    bmt = jnp.tile(bm, 8)
    b2t = jnp.tile(b2, 8)
    sx = _np_f32(_SX)
    pswap = _np_f32(_PSWAP)
    ppair = _np_f32(_PPAIR)

    hist = _sc_hist(dst3, nchunks)
    histp = hist.reshape(NC, NPAD8, 128)
    h1p = _tc_matmul1(x8, w1k)
    hs1p, dinvp = _tc_prep(h1p, histp)
    p1 = _sc_agg(hs1p.reshape(NPAD, H), src3, dst3, nchunks)
    hs2p = _tc_mid(p1.reshape(NC, NPAD8, 128), hs1p, dinvp, b1t, wmk)
    p2 = _sc_agg(hs2p.reshape(NPAD, H), src3, dst3, nchunks)
    hs3p = _tc_mid(p2.reshape(NC, NPAD8, 128), hs2p, dinvp, bmt, w2k)
    p3 = _sc_agg(hs3p.reshape(NPAD, H), src3, dst3, nchunks)
    out2 = _tc_final(p3.reshape(NC, NPAD8, 128), hs3p, dinvp, b2t,
                     sx, pswap, ppair)
    return out2.reshape(N, 2)
